# baseline (speedup 1.0000x reference)
import functools

import jax
import jax.numpy as jnp
from jax import lax
from jax.experimental import pallas as pl
from jax.experimental.pallas import tpu as pltpu
from jax.experimental.pallas import tpu_sc as plsc

N_USER = 10000
N_ITEM = 10000
E = 320000
D = 128

NC = 2
NS = 16
L = 16

CHUNK = 128
NUM_CHUNKS = E // CHUNK
CPT = NUM_CHUNKS // NS
EXTRA_TILES = NUM_CHUNKS - CPT * NS
K = 6
NB = CPT // K

N_NODES = N_USER
FIN_TILE_ROWS = 640
FIN_BLOCK = 80
LAST_ROWS = N_NODES - (NS - 1) * FIN_TILE_ROWS
NBLK_FULL = FIN_TILE_ROWS // FIN_BLOCK
NBLK_LAST = LAST_ROWS // FIN_BLOCK


def _sc_body(embed0, embed1, bias_hbm, src0, dst0, src1, dst1,
             out_user, out_item,
             acc, cnt, ixs0, ixs1, ixd0, ixd1, ixst, ixdt,
             rows0, rows1, ones, facc, fcnt, bias_v, sem_g, sem_is, sem_id):
    cid = lax.axis_index("c")
    sid = lax.axis_index("s")

    fin_base = sid * FIN_TILE_ROWS

    def per_tile_blocks(body):
        @pl.when(sid < NS - 1)
        def _():
            lax.fori_loop(0, NBLK_FULL, body, None)

        @pl.when(sid == NS - 1)
        def _():
            lax.fori_loop(0, NBLK_LAST, body, None)

    one_vec = jnp.ones((L,), jnp.float32)
    zero_vec = jnp.zeros((L,), jnp.float32)

    def zero_row(r, carry):
        for j in range(D // L):
            facc[r, pl.ds(j * L, L)] = zero_vec
        return carry

    lax.fori_loop(0, FIN_BLOCK, zero_row, None)
    for j in range(FIN_TILE_ROWS // L):
        fcnt[pl.ds(j * L, L)] = zero_vec
    for j in range(CHUNK // L):
        ones[pl.ds(j * L, L)] = one_vec

    def zero_block(b, carry):
        pltpu.sync_copy(facc, acc.at[pl.ds(fin_base + b * FIN_BLOCK,
                                           FIN_BLOCK)])
        return carry

    per_tile_blocks(zero_block)

    @pl.when(sid < NS - 1)
    def _():
        pltpu.sync_copy(fcnt, cnt.at[pl.ds(fin_base, FIN_TILE_ROWS)])

    @pl.when(sid == NS - 1)
    def _():
        pltpu.sync_copy(fcnt.at[pl.ds(0, LAST_ROWS)],
                        cnt.at[pl.ds(fin_base, LAST_ROWS)])

    plsc.subcore_barrier()

    ixs = (ixs0, ixs1)
    ixd = (ixd0, ixd1)
    rows = (rows0, rows1)

    def run_etype(embed_hbm, src_hbm, dst_hbm):
        cbase = sid * CPT

        def start_idxblk(p, bb):
            off = (cbase + p * K) * CHUNK
            pltpu.async_copy(src_hbm.at[pl.ds(off, K * CHUNK)], ixs[bb],
                             sem_is)
            pltpu.async_copy(dst_hbm.at[pl.ds(off, K * CHUNK)], ixd[bb],
                             sem_id)

        def wait_idxblk(bb):
            pltpu.make_async_copy(src_hbm.at[pl.ds(0, K * CHUNK)], ixs[bb],
                                  sem_is).wait()
            pltpu.make_async_copy(dst_hbm.at[pl.ds(0, K * CHUNK)], ixd[bb],
                                  sem_id).wait()

        def start_gather(bb, k, rb):
            pltpu.async_copy(
                embed_hbm.at[ixs[bb].at[pl.ds(k * CHUNK, CHUNK)]],
                rows[rb], sem_g)

        def wait_gather(bb, k, rb):
            pltpu.make_async_copy(
                embed_hbm.at[ixs[bb].at[pl.ds(k * CHUNK, CHUNK)]],
                rows[rb], sem_g).wait()

        start_idxblk(0, 0)
        wait_idxblk(0)
        start_gather(0, 0, 0)
        start_idxblk(1, 1)

        def outer(P, carry):
            for bp in range(2):
                p = 2 * P + bp
                B = bp
                for k in range(K):
                    rb = k % 2
                    nrb = 1 - rb
                    wait_gather(B, k, rb)
                    if k == K - 1:
                        wait_idxblk(1 - B)
                        start_gather(1 - B, 0, nrb)
                    else:
                        start_gather(B, k + 1, nrb)
                    dref = ixd[B].at[pl.ds(k * CHUNK, CHUNK)]
                    pltpu.sync_copy(rows[rb], acc.at[dref], add=True)
                    pltpu.sync_copy(ones, cnt.at[dref], add=True)
                start_idxblk(jnp.minimum(p + 2, NB - 1), B)
            return carry

        lax.fori_loop(0, NB // 2, outer, None)
        wait_gather(0, 0, 0)
        wait_idxblk(1)

        @pl.when(sid < EXTRA_TILES)
        def _():
            eoff = (CPT * NS + sid) * CHUNK
            pltpu.sync_copy(src_hbm.at[pl.ds(eoff, CHUNK)], ixst)
            pltpu.sync_copy(dst_hbm.at[pl.ds(eoff, CHUNK)], ixdt)
            pltpu.async_copy(embed_hbm.at[ixst], rows0, sem_g).wait()
            pltpu.sync_copy(rows0, acc.at[ixdt], add=True)
            pltpu.sync_copy(ones, cnt.at[ixdt], add=True)

    @pl.when(cid == 0)
    def _():
        run_etype(embed0, src0, dst0)

    @pl.when(cid == 1)
    def _():
        run_etype(embed1, src1, dst1)

    plsc.subcore_barrier()

    pltpu.sync_copy(bias_hbm, bias_v)

    @pl.when(sid < NS - 1)
    def _():
        pltpu.sync_copy(cnt.at[pl.ds(fin_base, FIN_TILE_ROWS)], fcnt)

    @pl.when(sid == NS - 1)
    def _():
        pltpu.sync_copy(cnt.at[pl.ds(fin_base, LAST_ROWS)],
                        fcnt.at[pl.ds(0, LAST_ROWS)])

    def run_finalize(out_hbm):
        def fin_block(b, carry):
            rbase = fin_base + b * FIN_BLOCK
            pltpu.sync_copy(acc.at[pl.ds(rbase, FIN_BLOCK)], facc)

            def fin_group(g, inner):
                cnt16 = fcnt[pl.ds(b * FIN_BLOCK + g * L, L)]
                scale16 = jnp.where(cnt16 > 0.0,
                                    1.0 / jnp.maximum(cnt16, 1.0),
                                    zero_vec)
                for j in range(L):
                    sv = jnp.broadcast_to(scale16[j], (L,))
                    r = g * L + j
                    for k in range(D // L):
                        sl = pl.ds(k * L, L)
                        facc[r, sl] = facc[r, sl] * sv + bias_v[sl]
                return inner

            lax.fori_loop(0, FIN_BLOCK // L, fin_group, None)
            pltpu.sync_copy(facc, out_hbm.at[pl.ds(rbase, FIN_BLOCK)])
            return carry

        per_tile_blocks(fin_block)

    @pl.when(cid == 0)
    def _():
        run_finalize(out_item)

    @pl.when(cid == 1)
    def _():
        run_finalize(out_user)


@jax.jit
def _rel_graph_conv(embed0, embed1, h_bias, src0, dst0, src1, dst1):
    mesh = plsc.VectorSubcoreMesh(core_axis_name="c", subcore_axis_name="s",
                                  num_cores=NC, num_subcores=NS)
    blk_t = pltpu.VMEM((K * CHUNK,), jnp.int32)
    kern = functools.partial(
        pl.kernel,
        out_type=(
            jax.ShapeDtypeStruct((N_USER, D), jnp.float32),
            jax.ShapeDtypeStruct((N_ITEM, D), jnp.float32),
        ),
        mesh=mesh,
        scratch_types=[
            pltpu.VMEM_SHARED((N_NODES, D), jnp.float32),
            pltpu.VMEM_SHARED((N_NODES,), jnp.float32),
            blk_t, blk_t,
            blk_t, blk_t,
            pltpu.VMEM((CHUNK,), jnp.int32),
            pltpu.VMEM((CHUNK,), jnp.int32),
            pltpu.VMEM((CHUNK, D), jnp.float32),
            pltpu.VMEM((CHUNK, D), jnp.float32),
            pltpu.VMEM((CHUNK,), jnp.float32),
            pltpu.VMEM((FIN_BLOCK, D), jnp.float32),
            pltpu.VMEM((FIN_TILE_ROWS,), jnp.float32),
            pltpu.VMEM((D,), jnp.float32),
            pltpu.SemaphoreType.DMA,
            pltpu.SemaphoreType.DMA,
            pltpu.SemaphoreType.DMA,
        ],
    )(_sc_body)
    return kern(embed0, embed1, h_bias, src0, dst0, src1, dst1)


def kernel(embed0, embed1, h_bias, src0, dst0, src1, dst1):
    return _rel_graph_conv(
        embed0.astype(jnp.float32),
        embed1.astype(jnp.float32),
        h_bias.astype(jnp.float32),
        src0.astype(jnp.int32),
        dst0.astype(jnp.int32),
        src1.astype(jnp.int32),
        dst1.astype(jnp.int32),
    )


# speedup vs baseline: 1.1526x; 1.1526x over previous
import functools

import jax
import jax.numpy as jnp
from jax import lax
from jax.experimental import pallas as pl
from jax.experimental.pallas import tpu as pltpu
from jax.experimental.pallas import tpu_sc as plsc

N_USER = 10000
N_ITEM = 10000
E = 320000
D = 128

NC = 2
NS = 16
L = 16

CHUNK = 128
NUM_CHUNKS = E // CHUNK
CPT = NUM_CHUNKS // NS
EXTRA_TILES = NUM_CHUNKS - CPT * NS
K = 6
NB = CPT // K

N_NODES = N_USER
FIN_TILE_ROWS = 640
FIN_BLOCK = 80
LAST_ROWS = N_NODES - (NS - 1) * FIN_TILE_ROWS
NBLK_FULL = FIN_TILE_ROWS // FIN_BLOCK
NBLK_LAST = LAST_ROWS // FIN_BLOCK


def _sc_body(embed0, embed1, bias_hbm, src0, dst0, src1, dst1,
             out_user, out_item,
             acc, cnt, ixs0, ixs1, ixd0, ixd1, ixst, ixdt,
             rows0, rows1, ones, facc, fcnt, bias_v,
             sem_g0, sem_g1, sem_is, sem_id):
    cid = lax.axis_index("c")
    sid = lax.axis_index("s")

    fin_base = sid * FIN_TILE_ROWS

    def per_tile_blocks(body):
        @pl.when(sid < NS - 1)
        def _():
            lax.fori_loop(0, NBLK_FULL, body, None)

        @pl.when(sid == NS - 1)
        def _():
            lax.fori_loop(0, NBLK_LAST, body, None)

    one_vec = jnp.ones((L,), jnp.float32)
    zero_vec = jnp.zeros((L,), jnp.float32)

    def zero_row(r, carry):
        for j in range(D // L):
            facc[r, pl.ds(j * L, L)] = zero_vec
        return carry

    lax.fori_loop(0, FIN_BLOCK, zero_row, None)
    for j in range(FIN_TILE_ROWS // L):
        fcnt[pl.ds(j * L, L)] = zero_vec
    for j in range(CHUNK // L):
        ones[pl.ds(j * L, L)] = one_vec

    def zero_block(b, carry):
        pltpu.sync_copy(facc, acc.at[pl.ds(fin_base + b * FIN_BLOCK,
                                           FIN_BLOCK)])
        return carry

    per_tile_blocks(zero_block)

    @pl.when(sid < NS - 1)
    def _():
        pltpu.sync_copy(fcnt, cnt.at[pl.ds(fin_base, FIN_TILE_ROWS)])

    @pl.when(sid == NS - 1)
    def _():
        pltpu.sync_copy(fcnt.at[pl.ds(0, LAST_ROWS)],
                        cnt.at[pl.ds(fin_base, LAST_ROWS)])

    plsc.subcore_barrier()

    ixs = (ixs0, ixs1)
    ixd = (ixd0, ixd1)
    rows = (rows0, rows1)
    sem_g = (sem_g0, sem_g1)

    def run_etype(embed_hbm, src_hbm, dst_hbm):
        cbase = sid * CPT

        def start_idxblk(p, bb):
            off = (cbase + p * K) * CHUNK
            pltpu.async_copy(src_hbm.at[pl.ds(off, K * CHUNK)], ixs[bb],
                             sem_is)
            pltpu.async_copy(dst_hbm.at[pl.ds(off, K * CHUNK)], ixd[bb],
                             sem_id)

        def wait_idxblk(bb):
            pltpu.make_async_copy(src_hbm.at[pl.ds(0, K * CHUNK)], ixs[bb],
                                  sem_is).wait()
            pltpu.make_async_copy(dst_hbm.at[pl.ds(0, K * CHUNK)], ixd[bb],
                                  sem_id).wait()

        def start_gather(bb, k, rb):
            pltpu.async_copy(
                embed_hbm.at[ixs[bb].at[pl.ds(k * CHUNK, CHUNK)]],
                rows[rb], sem_g[rb])

        def wait_gather(bb, k, rb):
            pltpu.make_async_copy(
                embed_hbm.at[ixs[bb].at[pl.ds(k * CHUNK, CHUNK)]],
                rows[rb], sem_g[rb]).wait()

        start_idxblk(0, 0)
        wait_idxblk(0)
        start_gather(0, 0, 0)
        start_idxblk(1, 1)

        def outer(P, carry):
            for bp in range(2):
                p = 2 * P + bp
                B = bp
                for k in range(K):
                    rb = k % 2
                    nrb = 1 - rb
                    if k == K - 1:
                        wait_idxblk(1 - B)
                        start_gather(1 - B, 0, nrb)
                    else:
                        start_gather(B, k + 1, nrb)
                    wait_gather(B, k, rb)
                    dref = ixd[B].at[pl.ds(k * CHUNK, CHUNK)]
                    pltpu.sync_copy(rows[rb], acc.at[dref], add=True)
                    pltpu.sync_copy(ones, cnt.at[dref], add=True)
                start_idxblk(jnp.minimum(p + 2, NB - 1), B)
            return carry

        lax.fori_loop(0, NB // 2, outer, None)
        wait_gather(0, 0, 0)
        wait_idxblk(1)

        @pl.when(sid < EXTRA_TILES)
        def _():
            eoff = (CPT * NS + sid) * CHUNK
            pltpu.sync_copy(src_hbm.at[pl.ds(eoff, CHUNK)], ixst)
            pltpu.sync_copy(dst_hbm.at[pl.ds(eoff, CHUNK)], ixdt)
            pltpu.async_copy(embed_hbm.at[ixst], rows0, sem_g0).wait()
            pltpu.sync_copy(rows0, acc.at[ixdt], add=True)
            pltpu.sync_copy(ones, cnt.at[ixdt], add=True)

    @pl.when(cid == 0)
    def _():
        run_etype(embed0, src0, dst0)

    @pl.when(cid == 1)
    def _():
        run_etype(embed1, src1, dst1)

    plsc.subcore_barrier()

    pltpu.sync_copy(bias_hbm, bias_v)

    @pl.when(sid < NS - 1)
    def _():
        pltpu.sync_copy(cnt.at[pl.ds(fin_base, FIN_TILE_ROWS)], fcnt)

    @pl.when(sid == NS - 1)
    def _():
        pltpu.sync_copy(cnt.at[pl.ds(fin_base, LAST_ROWS)],
                        fcnt.at[pl.ds(0, LAST_ROWS)])

    def run_finalize(out_hbm):
        def fin_block(b, carry):
            rbase = fin_base + b * FIN_BLOCK
            pltpu.sync_copy(acc.at[pl.ds(rbase, FIN_BLOCK)], facc)

            def fin_group(g, inner):
                cnt16 = fcnt[pl.ds(b * FIN_BLOCK + g * L, L)]
                scale16 = jnp.where(cnt16 > 0.0,
                                    1.0 / jnp.maximum(cnt16, 1.0),
                                    zero_vec)
                for j in range(L):
                    sv = jnp.broadcast_to(scale16[j], (L,))
                    r = g * L + j
                    for k in range(D // L):
                        sl = pl.ds(k * L, L)
                        facc[r, sl] = facc[r, sl] * sv + bias_v[sl]
                return inner

            lax.fori_loop(0, FIN_BLOCK // L, fin_group, None)
            pltpu.sync_copy(facc, out_hbm.at[pl.ds(rbase, FIN_BLOCK)])
            return carry

        per_tile_blocks(fin_block)

    @pl.when(cid == 0)
    def _():
        run_finalize(out_item)

    @pl.when(cid == 1)
    def _():
        run_finalize(out_user)


@jax.jit
def _rel_graph_conv(embed0, embed1, h_bias, src0, dst0, src1, dst1):
    mesh = plsc.VectorSubcoreMesh(core_axis_name="c", subcore_axis_name="s",
                                  num_cores=NC, num_subcores=NS)
    blk_t = pltpu.VMEM((K * CHUNK,), jnp.int32)
    kern = functools.partial(
        pl.kernel,
        out_type=(
            jax.ShapeDtypeStruct((N_USER, D), jnp.float32),
            jax.ShapeDtypeStruct((N_ITEM, D), jnp.float32),
        ),
        mesh=mesh,
        scratch_types=[
            pltpu.VMEM_SHARED((N_NODES, D), jnp.float32),
            pltpu.VMEM_SHARED((N_NODES,), jnp.float32),
            blk_t, blk_t,
            blk_t, blk_t,
            pltpu.VMEM((CHUNK,), jnp.int32),
            pltpu.VMEM((CHUNK,), jnp.int32),
            pltpu.VMEM((CHUNK, D), jnp.float32),
            pltpu.VMEM((CHUNK, D), jnp.float32),
            pltpu.VMEM((CHUNK,), jnp.float32),
            pltpu.VMEM((FIN_BLOCK, D), jnp.float32),
            pltpu.VMEM((FIN_TILE_ROWS,), jnp.float32),
            pltpu.VMEM((D,), jnp.float32),
            pltpu.SemaphoreType.DMA,
            pltpu.SemaphoreType.DMA,
            pltpu.SemaphoreType.DMA,
            pltpu.SemaphoreType.DMA,
        ],
    )(_sc_body)
    return kern(embed0, embed1, h_bias, src0, dst0, src1, dst1)


def kernel(embed0, embed1, h_bias, src0, dst0, src1, dst1):
    return _rel_graph_conv(
        embed0.astype(jnp.float32),
        embed1.astype(jnp.float32),
        h_bias.astype(jnp.float32),
        src0.astype(jnp.int32),
        dst0.astype(jnp.int32),
        src1.astype(jnp.int32),
        dst1.astype(jnp.int32),
    )
